# baseline (speedup 1.0000x reference)
import jax
import jax.numpy as jnp
from jax.experimental import pallas as pl
from jax.experimental.pallas import tpu as pltpu

_K = 16
_F = 14
_C = 2


def _mlp_block_kernel(w1b_ref, b1_ref, w2b_ref, b2_ref, x_ref, o_ref):
    x = x_ref[...]
    h = jnp.dot(x, w1b_ref[...], preferred_element_type=jnp.float32)
    h = jnp.maximum(h + b1_ref[...], 0.0)
    o = jnp.dot(h, w2b_ref[...], preferred_element_type=jnp.float32)
    o_ref[...] = o + b2_ref[...]


def kernel(x, w1, b1, w2, b2):
    B, F = x.shape
    R = B // _K
    KF = _K * _F
    KC = _K * _C

    xv = x.reshape(R, KF)

    eye = jnp.eye(_K, dtype=jnp.float32)
    w1b = (eye[:, None, :, None] * w1.T[None, :, None, :]).reshape(KF, KF)
    w2b = (eye[:, None, :, None] * w2.T[None, :, None, :]).reshape(KF, KC)
    b1t = jnp.tile(b1, _K).reshape(1, KF)
    b2t = jnp.tile(b2, _K).reshape(1, KC)

    tb = 2048
    grid = (R // tb,)

    ov = pl.pallas_call(
        _mlp_block_kernel,
        out_shape=jax.ShapeDtypeStruct((R, KC), jnp.float32),
        grid=grid,
        in_specs=[
            pl.BlockSpec((KF, KF), lambda i: (0, 0)),
            pl.BlockSpec((1, KF), lambda i: (0, 0)),
            pl.BlockSpec((KF, KC), lambda i: (0, 0)),
            pl.BlockSpec((1, KC), lambda i: (0, 0)),
            pl.BlockSpec((tb, KF), lambda i: (i, 0)),
        ],
        out_specs=pl.BlockSpec((tb, KC), lambda i: (i, 0)),
        compiler_params=pltpu.CompilerParams(
            dimension_semantics=("parallel",)),
    )(w1b, b1t, w2b, b2t, xv)

    return ov.reshape(B, _C)


# speedup vs baseline: 1.2708x; 1.2708x over previous
import jax
import jax.numpy as jnp
from jax import lax
from jax.experimental import pallas as pl
from jax.experimental.pallas import tpu as pltpu


def _mlp_kernel(w1_ref, b1_ref, w2_ref, b2_ref, x_ref, o_ref):
    x = x_ref[...]
    h = lax.dot_general(x, w1_ref[...], (((1,), (1,)), ((), ())),
                        preferred_element_type=jnp.float32)
    h = jnp.maximum(h + b1_ref[...], 0.0)
    o = lax.dot_general(h, w2_ref[...], (((1,), (1,)), ((), ())),
                        preferred_element_type=jnp.float32)
    o_ref[...] = o + b2_ref[...]


def kernel(x, w1, b1, w2, b2):
    B, F = x.shape
    C = w2.shape[0]
    b1r = b1.reshape(1, F)
    b2r = b2.reshape(1, C)

    tb = 8192
    grid = (B // tb,)

    out = pl.pallas_call(
        _mlp_kernel,
        out_shape=jax.ShapeDtypeStruct((B, C), jnp.float32),
        grid=grid,
        in_specs=[
            pl.BlockSpec((F, F), lambda i: (0, 0)),
            pl.BlockSpec((1, F), lambda i: (0, 0)),
            pl.BlockSpec((C, F), lambda i: (0, 0)),
            pl.BlockSpec((1, C), lambda i: (0, 0)),
            pl.BlockSpec((tb, F), lambda i: (i, 0)),
        ],
        out_specs=pl.BlockSpec((tb, C), lambda i: (i, 0)),
        compiler_params=pltpu.CompilerParams(
            dimension_semantics=("parallel",)),
    )(w1, b1r, w2, b2r, x)
    return out


# speedup vs baseline: 24.1614x; 19.0132x over previous
import jax
import jax.numpy as jnp
from jax.experimental import pallas as pl
from jax.experimental.pallas import tpu as pltpu


def _mlp_kernel(w1_ref, b1_ref, w2_ref, b2_ref, x_ref, o_ref):
    x = x_ref[...]
    h = jnp.dot(w1_ref[...], x, preferred_element_type=jnp.float32)
    h = jnp.maximum(h + b1_ref[...], 0.0)
    o = jnp.dot(w2_ref[...], h, preferred_element_type=jnp.float32)
    o_ref[...] = o + b2_ref[...]


def kernel(x, w1, b1, w2, b2):
    B, F = x.shape
    C = w2.shape[0]
    xt = x.T
    b1c = b1.reshape(F, 1)
    b2c = b2.reshape(C, 1)

    tb = 32768
    grid = (B // tb,)

    out_t = pl.pallas_call(
        _mlp_kernel,
        out_shape=jax.ShapeDtypeStruct((C, B), jnp.float32),
        grid=grid,
        in_specs=[
            pl.BlockSpec((F, F), lambda i: (0, 0)),
            pl.BlockSpec((F, 1), lambda i: (0, 0)),
            pl.BlockSpec((C, F), lambda i: (0, 0)),
            pl.BlockSpec((C, 1), lambda i: (0, 0)),
            pl.BlockSpec((F, tb), lambda i: (0, i)),
        ],
        out_specs=pl.BlockSpec((C, tb), lambda i: (0, i)),
        compiler_params=pltpu.CompilerParams(
            dimension_semantics=("parallel",)),
    )(w1, b1c, w2, b2c, xt)
    return out_t.T


# speedup vs baseline: 29.3472x; 1.2146x over previous
import jax
import jax.numpy as jnp
from jax.experimental import pallas as pl
from jax.experimental.pallas import tpu as pltpu


def _mlp_kernel(w1_ref, b1_ref, w2_ref, b2_ref, x_ref, o_ref):
    x = x_ref[...]
    h = jnp.dot(w1_ref[...], x, preferred_element_type=jnp.float32)
    h = jnp.maximum(h + b1_ref[...], 0.0)
    o = jnp.dot(w2_ref[...], h, preferred_element_type=jnp.float32)
    o_ref[...] = o + b2_ref[...]


def kernel(x, w1, b1, w2, b2):
    B, F = x.shape
    C = w2.shape[0]
    xt = x.T
    b1c = b1.reshape(F, 1)
    b2c = b2.reshape(C, 1)

    tb = 65536
    grid = (B // tb,)

    out_t = pl.pallas_call(
        _mlp_kernel,
        out_shape=jax.ShapeDtypeStruct((C, B), jnp.float32),
        grid=grid,
        in_specs=[
            pl.BlockSpec((F, F), lambda i: (0, 0)),
            pl.BlockSpec((F, 1), lambda i: (0, 0)),
            pl.BlockSpec((C, F), lambda i: (0, 0)),
            pl.BlockSpec((C, 1), lambda i: (0, 0)),
            pl.BlockSpec((F, tb), lambda i: (0, i)),
        ],
        out_specs=pl.BlockSpec((C, tb), lambda i: (0, i)),
        compiler_params=pltpu.CompilerParams(
            dimension_semantics=("parallel",)),
    )(w1, b1c, w2, b2c, xt)
    return out_t.T


# speedup vs baseline: 31.2946x; 1.0664x over previous
import jax
import jax.numpy as jnp
from jax.experimental import pallas as pl
from jax.experimental.pallas import tpu as pltpu


def _mlp_kernel(w1_ref, b1_ref, w2_ref, b2_ref, x_ref, o_ref):
    x = x_ref[...]
    h = jnp.dot(w1_ref[...], x, preferred_element_type=jnp.float32)
    h = jnp.maximum(h + b1_ref[...], 0.0)
    o = jnp.dot(w2_ref[...], h, preferred_element_type=jnp.float32)
    o_ref[...] = o + b2_ref[...]


def kernel(x, w1, b1, w2, b2):
    B, F = x.shape
    C = w2.shape[0]
    xt = x.T
    b1c = b1.reshape(F, 1)
    b2c = b2.reshape(C, 1)

    tb = 131072
    grid = (B // tb,)

    out_t = pl.pallas_call(
        _mlp_kernel,
        out_shape=jax.ShapeDtypeStruct((C, B), jnp.float32),
        grid=grid,
        in_specs=[
            pl.BlockSpec((F, F), lambda i: (0, 0)),
            pl.BlockSpec((F, 1), lambda i: (0, 0)),
            pl.BlockSpec((C, F), lambda i: (0, 0)),
            pl.BlockSpec((C, 1), lambda i: (0, 0)),
            pl.BlockSpec((F, tb), lambda i: (0, i)),
        ],
        out_specs=pl.BlockSpec((C, tb), lambda i: (0, i)),
        compiler_params=pltpu.CompilerParams(
            dimension_semantics=("parallel",)),
    )(w1, b1c, w2, b2c, xt)
    return out_t.T


# speedup vs baseline: 36.5823x; 1.1690x over previous
import jax
import jax.numpy as jnp
from jax.experimental import pallas as pl
from jax.experimental.pallas import tpu as pltpu


def _mlp_kernel(w1_ref, b1_ref, w2_ref, b2_ref, x_ref, o_ref):
    x = x_ref[...]
    h = jnp.dot(w1_ref[...], x, preferred_element_type=jnp.float32)
    h = jnp.maximum(h + b1_ref[...].T, 0.0)
    o = jnp.dot(w2_ref[...], h, preferred_element_type=jnp.float32)
    o_ref[...] = o + b2_ref[...].T


def kernel(x, w1, b1, w2, b2):
    B, F = x.shape
    C = w2.shape[0]
    xt = x.T
    b1c = b1.reshape(1, F)
    b2c = b2.reshape(1, C)

    tb = 131072
    grid = (B // tb,)

    out_t = pl.pallas_call(
        _mlp_kernel,
        out_shape=jax.ShapeDtypeStruct((C, B), jnp.float32),
        grid=grid,
        in_specs=[
            pl.BlockSpec((F, F), lambda i: (0, 0)),
            pl.BlockSpec((1, F), lambda i: (0, 0)),
            pl.BlockSpec((C, F), lambda i: (0, 0)),
            pl.BlockSpec((1, C), lambda i: (0, 0)),
            pl.BlockSpec((F, tb), lambda i: (0, i)),
        ],
        out_specs=pl.BlockSpec((C, tb), lambda i: (0, i)),
        compiler_params=pltpu.CompilerParams(
            dimension_semantics=("parallel",)),
    )(w1, b1c, w2, b2c, xt)
    return out_t.T
